# baseline (speedup 1.0000x reference)
import jax
import jax.numpy as jnp
from jax import lax
from jax.experimental import pallas as pl
from jax.experimental.pallas import tpu as pltpu

B, T = 2, 2048
D, H, KH = 768, 12, 6
DH = 64
E, TOPK, F = 8, 2, 512
NEG = -1e30
TB = 512
NTB = (B * T) // TB
CH = 512

_f32 = jnp.float32


def _ln_body(x, g, b):
    mu = jnp.mean(x, axis=-1, keepdims=True)
    xc = x - mu
    var = jnp.mean(xc * xc, axis=-1, keepdims=True)
    return xc * lax.rsqrt(var + 1e-5) * g + b


def _rot(t):
    half = t.shape[-1] // 2
    return jnp.concatenate([-t[:, half:], t[:, :half]], axis=-1)


def _a1_body(x_ref, g_ref, b_ref, wrpt_ref, cos_ref, sin_ref, h_ref, pooled_ref):
    x = x_ref[0]
    h = _ln_body(x, g_ref[0], b_ref[0])
    h_ref[0] = h
    xr = jnp.dot(h, wrpt_ref[...], preferred_element_type=_f32)
    xr = xr * cos_ref[...] + _rot(xr) * sin_ref[...]
    pooled_ref[0, 0, :] = jnp.sum(xr, axis=0) * (1.0 / T)


def _a2_body(pooled_ref, wrt_ref, rw_ref, ti_ref):
    logits = jnp.dot(pooled_ref[...], wrt_ref[...], preferred_element_type=_f32)
    it = lax.broadcasted_iota(jnp.int32, (B, H), 1)
    vs, ids = [], []
    l = logits
    for _ in range(KH):
        m = jnp.max(l, axis=1, keepdims=True)
        i = jnp.min(jnp.where(l == m, it, H), axis=1, keepdims=True)
        vs.append(m)
        ids.append(i)
        l = jnp.where(it == i, NEG, l)
    tv = jnp.concatenate(vs, axis=1)
    ex = jnp.exp(tv - tv[:, :1])
    w = ex / jnp.sum(ex, axis=1, keepdims=True)
    it128 = lax.broadcasted_iota(jnp.int32, (B, 128), 1)
    rw = jnp.zeros((B, 128), _f32)
    ti = jnp.zeros((B, 128), jnp.int32)
    for kk in range(KH):
        rw = jnp.where(it128 == kk, w[:, kk:kk + 1], rw)
        ti = jnp.where(it128 == kk, ids[kk], ti)
    rw_ref[...] = rw
    ti_ref[...] = ti


def _b_body(tif_ref, rwf_ref, h_ref, wq_ref, wk_ref, wv_ref, wo_ref,
            cos_ref, sin_ref, out_ref):
    b = pl.program_id(0)
    kk = pl.program_id(1)
    bf = jnp.bfloat16
    h = h_ref[0].astype(bf)
    cos = cos_ref[...]
    sin = sin_ref[...]
    q = jnp.dot(h, wq_ref[0].astype(bf), preferred_element_type=_f32)
    k = jnp.dot(h, wk_ref[0].astype(bf), preferred_element_type=_f32)
    v = jnp.dot(h, wv_ref[0].astype(bf), preferred_element_type=_f32).astype(bf)
    q = q * cos + _rot(q) * sin
    k = (k * cos + _rot(k) * sin).astype(bf)
    wgt = rwf_ref[b * KH + kk]

    @pl.when(kk == 0)
    def _():
        out_ref[0] = jnp.zeros((T, D), _f32)

    scale = DH ** -0.5
    wo = wo_ref[0].astype(bf)
    for i in range(T // CH):
        qc = (q[i * CH:(i + 1) * CH] * scale).astype(bf)
        s = lax.dot_general(qc, k, (((1,), (1,)), ((), ())),
                            preferred_element_type=_f32)
        rows = lax.broadcasted_iota(jnp.int32, (CH, T), 0) + i * CH
        cols = lax.broadcasted_iota(jnp.int32, (CH, T), 1)
        s = jnp.where(cols > rows, NEG, s)
        m = jnp.max(s, axis=1, keepdims=True)
        p = jnp.exp(s - m)
        p = (p / jnp.sum(p, axis=1, keepdims=True)).astype(bf)
        ctx = jnp.dot(p, v, preferred_element_type=_f32)
        oph = jnp.dot(ctx.astype(bf), wo, preferred_element_type=_f32)
        out_ref[0, i * CH:(i + 1) * CH, :] += oph * wgt


def _c0_body(x_ref, a_ref, g_ref, b_ref, wrt_ref, x1_ref, h2_ref, fw_ref):
    x1 = x_ref[...] + a_ref[...]
    x1_ref[...] = x1
    h2 = _ln_body(x1, g_ref[0], b_ref[0])
    h2_ref[...] = h2
    rl = jnp.dot(h2, wrt_ref[...], preferred_element_type=_f32)
    it = lax.broadcasted_iota(jnp.int32, (TB, E), 1)
    m1 = jnp.max(rl, axis=1, keepdims=True)
    i1 = jnp.min(jnp.where(rl == m1, it, E), axis=1, keepdims=True)
    rl2 = jnp.where(it == i1, NEG, rl)
    m2 = jnp.max(rl2, axis=1, keepdims=True)
    i2 = jnp.min(jnp.where(rl2 == m2, it, E), axis=1, keepdims=True)
    w1 = 1.0 / (1.0 + jnp.exp(m2 - m1))
    w2 = 1.0 - w1
    fw = jnp.where(it == i1, w1, 0.0) + jnp.where(it == i2, w2, 0.0)
    fw_ref[...] = fw


def _c1_body(h2_ref, w1_ref, w3_ref, w2_ref, fw_ref, x1_ref, out_ref):
    e = pl.program_id(1)

    @pl.when(e == 0)
    def _():
        out_ref[...] = x1_ref[...]

    bf = jnp.bfloat16
    h2 = h2_ref[...].astype(bf)
    h1 = jnp.dot(h2, w1_ref[0].astype(bf), preferred_element_type=_f32)
    h3 = jnp.dot(h2, w3_ref[0].astype(bf), preferred_element_type=_f32)
    he = (h1 * (1.0 / (1.0 + jnp.exp(-h1))) * h3).astype(bf)
    eo = jnp.dot(he, w2_ref[0].astype(bf), preferred_element_type=_f32)
    it = lax.broadcasted_iota(jnp.int32, (TB, E), 1)
    wcol = jnp.sum(jnp.where(it == e, fw_ref[...], 0.0), axis=1, keepdims=True)
    out_ref[...] += eo * wcol


def kernel(x, causal_mask, attention_mask, positions, ln1_g, ln1_b, ln2_g, ln2_b,
           Wrp, Wr, Wq, Wk, Wv, Wo, Wrouter, W1, W2, W3):
    half = DH // 2
    inv_freq = 1.0 / (10000.0 ** (jnp.arange(half, dtype=_f32) * 2.0 / DH))
    ang = positions.astype(_f32)[:, None] * inv_freq[None, :]
    cos = jnp.concatenate([jnp.cos(ang), jnp.cos(ang)], axis=-1)
    sin = jnp.concatenate([jnp.sin(ang), jnp.sin(ang)], axis=-1)

    g1 = ln1_g.reshape(1, D)
    b1 = ln1_b.reshape(1, D)
    g2 = ln2_g.reshape(1, D)
    b2 = ln2_b.reshape(1, D)

    h, pooled = pl.pallas_call(
        _a1_body,
        grid=(B,),
        in_specs=[
            pl.BlockSpec((1, T, D), lambda b: (b, 0, 0)),
            pl.BlockSpec((1, D), lambda b: (0, 0)),
            pl.BlockSpec((1, D), lambda b: (0, 0)),
            pl.BlockSpec((D, DH), lambda b: (0, 0)),
            pl.BlockSpec((T, DH), lambda b: (0, 0)),
            pl.BlockSpec((T, DH), lambda b: (0, 0)),
        ],
        out_specs=[
            pl.BlockSpec((1, T, D), lambda b: (b, 0, 0)),
            pl.BlockSpec((1, 1, DH), lambda b: (b, 0, 0)),
        ],
        out_shape=[
            jax.ShapeDtypeStruct((B, T, D), _f32),
            jax.ShapeDtypeStruct((B, 1, DH), _f32),
        ],
    )(x, g1, b1, Wrp.T, cos, sin)

    rw_pad, ti_pad = pl.pallas_call(
        _a2_body,
        out_shape=[
            jax.ShapeDtypeStruct((B, 128), _f32),
            jax.ShapeDtypeStruct((B, 128), jnp.int32),
        ],
    )(pooled.reshape(B, DH), Wr.T)

    tif = ti_pad[:, :KH].reshape(B * KH)
    rwf = rw_pad[:, :KH].reshape(B * KH)

    attn = pl.pallas_call(
        _b_body,
        grid_spec=pltpu.PrefetchScalarGridSpec(
            num_scalar_prefetch=2,
            grid=(B, KH),
            in_specs=[
                pl.BlockSpec((1, T, D), lambda b, k, tif, rwf: (b, 0, 0)),
                pl.BlockSpec((1, D, DH), lambda b, k, tif, rwf: (tif[b * KH + k], 0, 0)),
                pl.BlockSpec((1, D, DH), lambda b, k, tif, rwf: (tif[b * KH + k], 0, 0)),
                pl.BlockSpec((1, D, DH), lambda b, k, tif, rwf: (tif[b * KH + k], 0, 0)),
                pl.BlockSpec((1, DH, D), lambda b, k, tif, rwf: (tif[b * KH + k], 0, 0)),
                pl.BlockSpec((T, DH), lambda b, k, tif, rwf: (0, 0)),
                pl.BlockSpec((T, DH), lambda b, k, tif, rwf: (0, 0)),
            ],
            out_specs=pl.BlockSpec((1, T, D), lambda b, k, tif, rwf: (b, 0, 0)),
        ),
        out_shape=jax.ShapeDtypeStruct((B, T, D), _f32),
        compiler_params=pltpu.CompilerParams(
            dimension_semantics=("arbitrary", "arbitrary")),
    )(tif, rwf, h, Wq, Wk, Wv, Wo, cos, sin)

    x2 = x.reshape(B * T, D)
    a2 = attn.reshape(B * T, D)
    x1, h2, fw = pl.pallas_call(
        _c0_body,
        grid=(NTB,),
        in_specs=[
            pl.BlockSpec((TB, D), lambda t: (t, 0)),
            pl.BlockSpec((TB, D), lambda t: (t, 0)),
            pl.BlockSpec((1, D), lambda t: (0, 0)),
            pl.BlockSpec((1, D), lambda t: (0, 0)),
            pl.BlockSpec((D, E), lambda t: (0, 0)),
        ],
        out_specs=[
            pl.BlockSpec((TB, D), lambda t: (t, 0)),
            pl.BlockSpec((TB, D), lambda t: (t, 0)),
            pl.BlockSpec((TB, E), lambda t: (t, 0)),
        ],
        out_shape=[
            jax.ShapeDtypeStruct((B * T, D), _f32),
            jax.ShapeDtypeStruct((B * T, D), _f32),
            jax.ShapeDtypeStruct((B * T, E), _f32),
        ],
    )(x2, a2, g2, b2, Wrouter.T)

    out = pl.pallas_call(
        _c1_body,
        grid=(NTB, E),
        in_specs=[
            pl.BlockSpec((TB, D), lambda t, e: (t, 0)),
            pl.BlockSpec((1, D, F), lambda t, e: (e, 0, 0)),
            pl.BlockSpec((1, D, F), lambda t, e: (e, 0, 0)),
            pl.BlockSpec((1, F, D), lambda t, e: (e, 0, 0)),
            pl.BlockSpec((TB, E), lambda t, e: (t, 0)),
            pl.BlockSpec((TB, D), lambda t, e: (t, 0)),
        ],
        out_specs=pl.BlockSpec((TB, D), lambda t, e: (t, 0)),
        out_shape=jax.ShapeDtypeStruct((B * T, D), _f32),
        compiler_params=pltpu.CompilerParams(
            dimension_semantics=("arbitrary", "arbitrary")),
    )(h2, W1, W3, W2, fw, x1)

    return out.reshape(B, T, D)


# speedup vs baseline: 1.0928x; 1.0928x over previous
import jax
import jax.numpy as jnp
from jax import lax
from jax.experimental import pallas as pl
from jax.experimental.pallas import tpu as pltpu

B, T = 2, 2048
D, H, KH = 768, 12, 6
DH = 64
E, TOPK, F = 8, 2, 512
NEG = -1e30
TB = 512
NTB = (B * T) // TB
CH = 512

_f32 = jnp.float32
_bf = jnp.bfloat16


def _ln_body(x, g, b):
    mu = jnp.mean(x, axis=-1, keepdims=True)
    xc = x - mu
    var = jnp.mean(xc * xc, axis=-1, keepdims=True)
    return xc * lax.rsqrt(var + 1e-5) * g + b


def _rot(t):
    half = t.shape[-1] // 2
    return jnp.concatenate([-t[:, half:], t[:, :half]], axis=-1)


def _bp_body(x_ref, g_ref, b_ref, wrpt_ref, wr_ref, wq_ref, wk_ref, wv_ref,
             wo_ref, cos_ref, sin_ref, out_ref, hbf_ref, ti_ref, tw_ref):
    kk = pl.program_id(1)
    cos = cos_ref[...]
    sin = sin_ref[...]

    @pl.when(kk == 0)
    def _():
        pooled = jnp.zeros((1, DH), _f32)
        for c in range(T // CH):
            sl = slice(c * CH, (c + 1) * CH)
            hc = _ln_body(x_ref[0, sl, :], g_ref[0], b_ref[0])
            hbf_ref[sl, :] = hc.astype(_bf)
            xr = jnp.dot(hc, wrpt_ref[...], preferred_element_type=_f32)
            xr = xr * cos[sl] + _rot(xr) * sin[sl]
            pooled = pooled + jnp.sum(xr, axis=0, keepdims=True)
        logits = jnp.dot(pooled * (1.0 / T), wr_ref[...],
                         preferred_element_type=_f32)
        it = lax.broadcasted_iota(jnp.int32, (1, H), 1)
        l = logits
        tvs, tis = [], []
        for _ in range(KH):
            m = jnp.max(l)
            i = jnp.min(jnp.where(l == m, it, H))
            tvs.append(m)
            tis.append(i)
            l = jnp.where(it == i, NEG, l)
        exps = [jnp.exp(v - tvs[0]) for v in tvs]
        denom = exps[0]
        for e_ in exps[1:]:
            denom = denom + e_
        for j in range(KH):
            ti_ref[j] = tis[j]
            tw_ref[j] = exps[j] / denom
        out_ref[0] = x_ref[0]

    idx = ti_ref[kk]
    wgt = tw_ref[kk]
    hbf = hbf_ref[...]
    scale = DH ** -0.5
    q = jnp.dot(hbf, wq_ref[idx], preferred_element_type=_f32)
    k = jnp.dot(hbf, wk_ref[idx], preferred_element_type=_f32)
    v = jnp.dot(hbf, wv_ref[idx], preferred_element_type=_f32).astype(_bf)
    q = q * cos + _rot(q) * sin
    k = (k * cos + _rot(k) * sin).astype(_bf)
    wo = wo_ref[idx]
    for i in range(T // CH):
        qc = (q[i * CH:(i + 1) * CH] * scale).astype(_bf)
        s = lax.dot_general(qc, k, (((1,), (1,)), ((), ())),
                            preferred_element_type=_f32)
        rows = lax.broadcasted_iota(jnp.int32, (CH, T), 0) + i * CH
        cols = lax.broadcasted_iota(jnp.int32, (CH, T), 1)
        s = jnp.where(cols > rows, NEG, s)
        m = jnp.max(s, axis=1, keepdims=True)
        p = jnp.exp(s - m)
        p = (p / jnp.sum(p, axis=1, keepdims=True)).astype(_bf)
        ctx = jnp.dot(p, v, preferred_element_type=_f32)
        oph = jnp.dot(ctx.astype(_bf), wo, preferred_element_type=_f32)
        out_ref[0, i * CH:(i + 1) * CH, :] += oph * wgt


def _c0_body(x1_ref, g_ref, b_ref, wrt_ref, h2_ref, fw_ref):
    h2 = _ln_body(x1_ref[...], g_ref[0], b_ref[0])
    h2_ref[...] = h2.astype(_bf)
    rl = jnp.dot(h2, wrt_ref[...], preferred_element_type=_f32)
    it = lax.broadcasted_iota(jnp.int32, (TB, E), 1)
    m1 = jnp.max(rl, axis=1, keepdims=True)
    i1 = jnp.min(jnp.where(rl == m1, it, E), axis=1, keepdims=True)
    rl2 = jnp.where(it == i1, NEG, rl)
    m2 = jnp.max(rl2, axis=1, keepdims=True)
    i2 = jnp.min(jnp.where(rl2 == m2, it, E), axis=1, keepdims=True)
    w1 = 1.0 / (1.0 + jnp.exp(m2 - m1))
    w2 = 1.0 - w1
    fw = jnp.where(it == i1, w1, 0.0) + jnp.where(it == i2, w2, 0.0)
    fw_ref[...] = fw


def _c1_body(h2_ref, w1_ref, w3_ref, w2_ref, fw_ref, x1_ref, out_ref):
    e = pl.program_id(1)

    @pl.when(e == 0)
    def _():
        out_ref[...] = x1_ref[...]

    h2 = h2_ref[...]
    h1 = jnp.dot(h2, w1_ref[0], preferred_element_type=_f32)
    h3 = jnp.dot(h2, w3_ref[0], preferred_element_type=_f32)
    he = (h1 * (1.0 / (1.0 + jnp.exp(-h1))) * h3).astype(_bf)
    eo = jnp.dot(he, w2_ref[0], preferred_element_type=_f32)
    it = lax.broadcasted_iota(jnp.int32, (TB, E), 1)
    wcol = jnp.sum(jnp.where(it == e, fw_ref[...], 0.0), axis=1, keepdims=True)
    out_ref[...] += eo * wcol


def kernel(x, causal_mask, attention_mask, positions, ln1_g, ln1_b, ln2_g, ln2_b,
           Wrp, Wr, Wq, Wk, Wv, Wo, Wrouter, W1, W2, W3):
    half = DH // 2
    inv_freq = 1.0 / (10000.0 ** (jnp.arange(half, dtype=_f32) * 2.0 / DH))
    ang = positions.astype(_f32)[:, None] * inv_freq[None, :]
    cos = jnp.concatenate([jnp.cos(ang), jnp.cos(ang)], axis=-1)
    sin = jnp.concatenate([jnp.sin(ang), jnp.sin(ang)], axis=-1)

    g1 = ln1_g.reshape(1, D)
    b1 = ln1_b.reshape(1, D)
    g2 = ln2_g.reshape(1, D)
    b2 = ln2_b.reshape(1, D)

    x1 = pl.pallas_call(
        _bp_body,
        grid=(B, KH),
        in_specs=[
            pl.BlockSpec((1, T, D), lambda b, k: (b, 0, 0)),
            pl.BlockSpec((1, D), lambda b, k: (0, 0)),
            pl.BlockSpec((1, D), lambda b, k: (0, 0)),
            pl.BlockSpec((D, DH), lambda b, k: (0, 0)),
            pl.BlockSpec((DH, H), lambda b, k: (0, 0)),
            pl.BlockSpec((H, D, DH), lambda b, k: (0, 0, 0)),
            pl.BlockSpec((H, D, DH), lambda b, k: (0, 0, 0)),
            pl.BlockSpec((H, D, DH), lambda b, k: (0, 0, 0)),
            pl.BlockSpec((H, DH, D), lambda b, k: (0, 0, 0)),
            pl.BlockSpec((T, DH), lambda b, k: (0, 0)),
            pl.BlockSpec((T, DH), lambda b, k: (0, 0)),
        ],
        out_specs=pl.BlockSpec((1, T, D), lambda b, k: (b, 0, 0)),
        out_shape=jax.ShapeDtypeStruct((B, T, D), _f32),
        scratch_shapes=[pltpu.VMEM((T, D), _bf),
                        pltpu.SMEM((KH,), jnp.int32),
                        pltpu.SMEM((KH,), _f32)],
        compiler_params=pltpu.CompilerParams(
            dimension_semantics=("arbitrary", "arbitrary")),
    )(x, g1, b1, Wrp.T, Wr.T, Wq.astype(_bf), Wk.astype(_bf), Wv.astype(_bf),
      Wo.astype(_bf), cos, sin)

    x12 = x1.reshape(B * T, D)
    h2, fw = pl.pallas_call(
        _c0_body,
        grid=(NTB,),
        in_specs=[
            pl.BlockSpec((TB, D), lambda t: (t, 0)),
            pl.BlockSpec((1, D), lambda t: (0, 0)),
            pl.BlockSpec((1, D), lambda t: (0, 0)),
            pl.BlockSpec((D, E), lambda t: (0, 0)),
        ],
        out_specs=[
            pl.BlockSpec((TB, D), lambda t: (t, 0)),
            pl.BlockSpec((TB, E), lambda t: (t, 0)),
        ],
        out_shape=[
            jax.ShapeDtypeStruct((B * T, D), _bf),
            jax.ShapeDtypeStruct((B * T, E), _f32),
        ],
    )(x12, g2, b2, Wrouter.T)

    out = pl.pallas_call(
        _c1_body,
        grid=(NTB, E),
        in_specs=[
            pl.BlockSpec((TB, D), lambda t, e: (t, 0)),
            pl.BlockSpec((1, D, F), lambda t, e: (e, 0, 0)),
            pl.BlockSpec((1, D, F), lambda t, e: (e, 0, 0)),
            pl.BlockSpec((1, F, D), lambda t, e: (e, 0, 0)),
            pl.BlockSpec((TB, E), lambda t, e: (t, 0)),
            pl.BlockSpec((TB, D), lambda t, e: (t, 0)),
        ],
        out_specs=pl.BlockSpec((TB, D), lambda t, e: (t, 0)),
        out_shape=jax.ShapeDtypeStruct((B * T, D), _f32),
        compiler_params=pltpu.CompilerParams(
            dimension_semantics=("arbitrary", "arbitrary")),
    )(h2, W1.astype(_bf), W3.astype(_bf), W2.astype(_bf), fw, x12)

    return out.reshape(B, T, D)


# speedup vs baseline: 1.0933x; 1.0005x over previous
import jax
import jax.numpy as jnp
from jax import lax
from jax.experimental import pallas as pl
from jax.experimental.pallas import tpu as pltpu

B, T = 2, 2048
D, H, KH = 768, 12, 6
DH = 64
E, TOPK, F = 8, 2, 512
NEG = -1e30
TB = 512
NTB = (B * T) // TB
CH = 512

_f32 = jnp.float32
_bf = jnp.bfloat16


def _ln_body(x, g, b):
    mu = jnp.mean(x, axis=-1, keepdims=True)
    xc = x - mu
    var = jnp.mean(xc * xc, axis=-1, keepdims=True)
    return xc * lax.rsqrt(var + 1e-5) * g + b


def _rot(t):
    half = t.shape[-1] // 2
    return jnp.concatenate([-t[:, half:], t[:, :half]], axis=-1)


def _bp_body(x_ref, g1_ref, b1_ref, wrpt_ref, wr_ref,
             wq_ref, wk_ref, wv_ref, wo_ref, cos_ref, sin_ref,
             x1_ref, hbf_ref, ti_ref, tw_ref):
    kk = pl.program_id(1)
    cos = cos_ref[...]
    sin = sin_ref[...]

    @pl.when(kk == 0)
    def _():
        pooled = jnp.zeros((1, DH), _f32)
        for c in range(T // CH):
            sl = slice(c * CH, (c + 1) * CH)
            hc = _ln_body(x_ref[0, sl, :], g1_ref[0], b1_ref[0])
            hbf_ref[sl, :] = hc
            xr = jnp.dot(hc, wrpt_ref[...], preferred_element_type=_f32)
            xr = xr * cos[sl] + _rot(xr) * sin[sl]
            pooled = pooled + jnp.sum(xr, axis=0, keepdims=True)
        logits = jnp.dot(pooled * (1.0 / T), wr_ref[...],
                         preferred_element_type=_f32)
        it = lax.broadcasted_iota(jnp.int32, (1, H), 1)
        l = logits
        tvs, tis = [], []
        for _ in range(KH):
            m = jnp.max(l)
            i = jnp.min(jnp.where(l == m, it, H))
            tvs.append(m)
            tis.append(i)
            l = jnp.where(it == i, NEG, l)
        exps = [jnp.exp(v - tvs[0]) for v in tvs]
        denom = exps[0]
        for e_ in exps[1:]:
            denom = denom + e_
        for j in range(KH):
            ti_ref[j] = tis[j]
            tw_ref[j] = exps[j] / denom
        x1_ref[0] = x_ref[0]

    idx = ti_ref[kk]
    wgt = tw_ref[kk]
    hbf = hbf_ref[...]
    scale = DH ** -0.5
    q = jnp.dot(hbf, wq_ref[idx], preferred_element_type=_f32)
    k = jnp.dot(hbf, wk_ref[idx], preferred_element_type=_f32)
    v = jnp.dot(hbf, wv_ref[idx], preferred_element_type=_f32)
    q = q * cos + _rot(q) * sin
    k = k * cos + _rot(k) * sin
    wo = wo_ref[idx]
    for i in range(T // CH):
        qc = q[i * CH:(i + 1) * CH] * scale
        s = lax.dot_general(qc, k, (((1,), (1,)), ((), ())),
                            preferred_element_type=_f32)
        rows = lax.broadcasted_iota(jnp.int32, (CH, T), 0) + i * CH
        cols = lax.broadcasted_iota(jnp.int32, (CH, T), 1)
        s = jnp.where(cols > rows, NEG, s)
        m = jnp.max(s, axis=1, keepdims=True)
        p = jnp.exp(s - m)
        p = p / jnp.sum(p, axis=1, keepdims=True)
        ctx = jnp.dot(p, v, preferred_element_type=_f32)
        oph = jnp.dot(ctx, wo, preferred_element_type=_f32)
        x1_ref[0, i * CH:(i + 1) * CH, :] += oph * wgt


def _c0_body(x1_ref, g_ref, b_ref, wrt_ref, h2_ref, fw_ref):
    h2 = _ln_body(x1_ref[...], g_ref[0], b_ref[0])
    h2_ref[...] = h2.astype(_bf)
    rl = jnp.dot(h2, wrt_ref[...], preferred_element_type=_f32)
    it = lax.broadcasted_iota(jnp.int32, (TB, E), 1)
    m1 = jnp.max(rl, axis=1, keepdims=True)
    i1 = jnp.min(jnp.where(rl == m1, it, E), axis=1, keepdims=True)
    rl2 = jnp.where(it == i1, NEG, rl)
    m2 = jnp.max(rl2, axis=1, keepdims=True)
    i2 = jnp.min(jnp.where(rl2 == m2, it, E), axis=1, keepdims=True)
    w1 = 1.0 / (1.0 + jnp.exp(m2 - m1))
    w2 = 1.0 - w1
    fw_ref[...] = jnp.where(it == i1, w1, 0.0) + jnp.where(it == i2, w2, 0.0)


def _c1_body(h2_ref, w1_ref, w3_ref, w2_ref, fw_ref, x1_ref, out_ref):
    e = pl.program_id(1)

    @pl.when(e == 0)
    def _():
        out_ref[...] = x1_ref[...]

    h2 = h2_ref[...]
    h1 = jnp.dot(h2, w1_ref[e], preferred_element_type=_f32)
    h3 = jnp.dot(h2, w3_ref[e], preferred_element_type=_f32)
    he = (h1 * (1.0 / (1.0 + jnp.exp(-h1))) * h3).astype(_bf)
    eo = jnp.dot(he, w2_ref[e], preferred_element_type=_f32)
    it = lax.broadcasted_iota(jnp.int32, (TB, E), 1)
    wcol = jnp.sum(jnp.where(it == e, fw_ref[...], 0.0), axis=1, keepdims=True)
    out_ref[...] += eo * wcol


def kernel(x, causal_mask, attention_mask, positions, ln1_g, ln1_b, ln2_g, ln2_b,
           Wrp, Wr, Wq, Wk, Wv, Wo, Wrouter, W1, W2, W3):
    half = DH // 2
    inv_freq = 1.0 / (10000.0 ** (jnp.arange(half, dtype=_f32) * 2.0 / DH))
    ang = positions.astype(_f32)[:, None] * inv_freq[None, :]
    cos = jnp.concatenate([jnp.cos(ang), jnp.cos(ang)], axis=-1)
    sin = jnp.concatenate([jnp.sin(ang), jnp.sin(ang)], axis=-1)

    g1 = ln1_g.reshape(1, D)
    b1 = ln1_b.reshape(1, D)
    g2 = ln2_g.reshape(1, D)
    b2 = ln2_b.reshape(1, D)

    full2 = lambda b, k: (0, 0)
    full3 = lambda b, k: (0, 0, 0)
    blk = lambda b, k: (b, 0, 0)
    x1 = pl.pallas_call(
        _bp_body,
        grid=(B, KH),
        in_specs=[
            pl.BlockSpec((1, T, D), blk),
            pl.BlockSpec((1, D), full2),
            pl.BlockSpec((1, D), full2),
            pl.BlockSpec((D, DH), full2),
            pl.BlockSpec((DH, H), full2),
            pl.BlockSpec((H, D, DH), full3),
            pl.BlockSpec((H, D, DH), full3),
            pl.BlockSpec((H, D, DH), full3),
            pl.BlockSpec((H, DH, D), full3),
            pl.BlockSpec((T, DH), full2),
            pl.BlockSpec((T, DH), full2),
        ],
        out_specs=pl.BlockSpec((1, T, D), blk),
        out_shape=jax.ShapeDtypeStruct((B, T, D), _f32),
        scratch_shapes=[pltpu.VMEM((T, D), _f32),
                        pltpu.SMEM((KH,), jnp.int32),
                        pltpu.SMEM((KH,), _f32)],
        compiler_params=pltpu.CompilerParams(
            dimension_semantics=("arbitrary", "arbitrary")),
    )(x, g1, b1, Wrp.T, Wr.T, Wq, Wk, Wv, Wo, cos, sin)

    x12r = x1.reshape(B * T, D)
    h2, fw = pl.pallas_call(
        _c0_body,
        grid=(NTB,),
        in_specs=[
            pl.BlockSpec((TB, D), lambda t: (t, 0)),
            pl.BlockSpec((1, D), lambda t: (0, 0)),
            pl.BlockSpec((1, D), lambda t: (0, 0)),
            pl.BlockSpec((D, E), lambda t: (0, 0)),
        ],
        out_specs=[
            pl.BlockSpec((TB, D), lambda t: (t, 0)),
            pl.BlockSpec((TB, E), lambda t: (t, 0)),
        ],
        out_shape=[
            jax.ShapeDtypeStruct((B * T, D), _bf),
            jax.ShapeDtypeStruct((B * T, E), _f32),
        ],
    )(x12r, g2, b2, Wrouter.T)

    x12 = x1.reshape(B * T, D)
    out = pl.pallas_call(
        _c1_body,
        grid=(NTB, E),
        in_specs=[
            pl.BlockSpec((TB, D), lambda t, e: (t, 0)),
            pl.BlockSpec((E, D, F), lambda t, e: (0, 0, 0)),
            pl.BlockSpec((E, D, F), lambda t, e: (0, 0, 0)),
            pl.BlockSpec((E, F, D), lambda t, e: (0, 0, 0)),
            pl.BlockSpec((TB, E), lambda t, e: (t, 0)),
            pl.BlockSpec((TB, D), lambda t, e: (t, 0)),
        ],
        out_specs=pl.BlockSpec((TB, D), lambda t, e: (t, 0)),
        out_shape=jax.ShapeDtypeStruct((B * T, D), _f32),
        compiler_params=pltpu.CompilerParams(
            dimension_semantics=("arbitrary", "arbitrary")),
    )(h2, W1.astype(_bf), W3.astype(_bf), W2.astype(_bf), fw, x12)

    return out.reshape(B, T, D)


# speedup vs baseline: 1.2260x; 1.1214x over previous
import jax
import jax.numpy as jnp
from jax import lax
from jax.experimental import pallas as pl
from jax.experimental.pallas import tpu as pltpu

B, T = 2, 2048
D, H, KH = 768, 12, 6
DH = 64
E, TOPK, F = 8, 2, 512
NEG = -1e30
TB = 512
NTB = (B * T) // TB
CH = 512

_f32 = jnp.float32
_bf = jnp.bfloat16


def _ln_body(x, g, b):
    mu = jnp.mean(x, axis=-1, keepdims=True)
    xc = x - mu
    var = jnp.mean(xc * xc, axis=-1, keepdims=True)
    return xc * lax.rsqrt(var + 1e-5) * g + b


def _rot(t):
    half = t.shape[-1] // 2
    return jnp.concatenate([-t[:, half:], t[:, :half]], axis=-1)


def _bp_body(x_ref, g1_ref, b1_ref, wrpt_ref, wr_ref,
             wq_ref, wk_ref, wv_ref, wo_ref, cos_ref, sin_ref,
             x1_ref, hbf_ref, ti_ref, tw_ref):
    kk = pl.program_id(1)
    cos = cos_ref[...]
    sin = sin_ref[...]

    @pl.when(kk == 0)
    def _():
        pooled = jnp.zeros((1, DH), _f32)
        for c in range(T // CH):
            sl = slice(c * CH, (c + 1) * CH)
            hc = _ln_body(x_ref[0, sl, :], g1_ref[0], b1_ref[0])
            hbf_ref[sl, :] = hc
            xr = jnp.dot(hc, wrpt_ref[...], preferred_element_type=_f32)
            xr = xr * cos[sl] + _rot(xr) * sin[sl]
            pooled = pooled + jnp.sum(xr, axis=0, keepdims=True)
        logits = jnp.dot(pooled * (1.0 / T), wr_ref[...],
                         preferred_element_type=_f32)
        it = lax.broadcasted_iota(jnp.int32, (1, H), 1)
        l = logits
        tvs, tis = [], []
        for _ in range(KH):
            m = jnp.max(l)
            i = jnp.min(jnp.where(l == m, it, H))
            tvs.append(m)
            tis.append(i)
            l = jnp.where(it == i, NEG, l)
        exps = [jnp.exp(v - tvs[0]) for v in tvs]
        denom = exps[0]
        for e_ in exps[1:]:
            denom = denom + e_
        for j in range(KH):
            ti_ref[j] = tis[j]
            tw_ref[j] = exps[j] / denom
        x1_ref[0] = x_ref[0]

    idx = ti_ref[kk]
    wgt = tw_ref[kk]
    hbf = hbf_ref[...]
    scale = DH ** -0.5
    q = jnp.dot(hbf, wq_ref[idx], preferred_element_type=_f32)
    k = jnp.dot(hbf, wk_ref[idx], preferred_element_type=_f32)
    v = jnp.dot(hbf, wv_ref[idx], preferred_element_type=_f32)
    q = q * cos + _rot(q) * sin
    k = k * cos + _rot(k) * sin
    wo = wo_ref[idx]
    vaug = jnp.concatenate([v, jnp.ones((T, DH), _f32)], axis=1)
    tri = (lax.broadcasted_iota(jnp.int32, (CH, CH), 1)
           > lax.broadcasted_iota(jnp.int32, (CH, CH), 0))
    for i in range(T // CH):
        kl = (i + 1) * CH
        qc = q[i * CH:kl] * scale
        s = lax.dot_general(qc, k[:kl], (((1,), (1,)), ((), ())),
                            preferred_element_type=_f32)
        pd = jnp.where(tri, 0.0, jnp.exp(s[:, i * CH:kl]))
        if i == 0:
            p = pd
        else:
            p = jnp.concatenate([jnp.exp(s[:, :i * CH]), pd], axis=1)
        ctxa = jnp.dot(p, vaug[:kl], preferred_element_type=_f32)
        ctx = ctxa[:, :DH] / ctxa[:, DH:DH + 1]
        oph = jnp.dot(ctx, wo, preferred_element_type=_f32)
        x1_ref[0, i * CH:kl, :] += oph * wgt


def _c0_body(x1_ref, g_ref, b_ref, wrt_ref, h2_ref, fw_ref):
    h2 = _ln_body(x1_ref[...], g_ref[0], b_ref[0])
    h2_ref[...] = h2.astype(_bf)
    rl = jnp.dot(h2, wrt_ref[...], preferred_element_type=_f32)
    it = lax.broadcasted_iota(jnp.int32, (TB, E), 1)
    m1 = jnp.max(rl, axis=1, keepdims=True)
    i1 = jnp.min(jnp.where(rl == m1, it, E), axis=1, keepdims=True)
    rl2 = jnp.where(it == i1, NEG, rl)
    m2 = jnp.max(rl2, axis=1, keepdims=True)
    i2 = jnp.min(jnp.where(rl2 == m2, it, E), axis=1, keepdims=True)
    w1 = 1.0 / (1.0 + jnp.exp(m2 - m1))
    w2 = 1.0 - w1
    fw_ref[...] = jnp.where(it == i1, w1, 0.0) + jnp.where(it == i2, w2, 0.0)


def _c1_body(h2_ref, w1_ref, w3_ref, w2_ref, fw_ref, x1_ref, out_ref):
    e = pl.program_id(1)

    @pl.when(e == 0)
    def _():
        out_ref[...] = x1_ref[...]

    h2 = h2_ref[...]
    h1 = jnp.dot(h2, w1_ref[e], preferred_element_type=_f32)
    h3 = jnp.dot(h2, w3_ref[e], preferred_element_type=_f32)
    he = (h1 * (1.0 / (1.0 + jnp.exp(-h1))) * h3).astype(_bf)
    eo = jnp.dot(he, w2_ref[e], preferred_element_type=_f32)
    it = lax.broadcasted_iota(jnp.int32, (TB, E), 1)
    wcol = jnp.sum(jnp.where(it == e, fw_ref[...], 0.0), axis=1, keepdims=True)
    out_ref[...] += eo * wcol


def kernel(x, causal_mask, attention_mask, positions, ln1_g, ln1_b, ln2_g, ln2_b,
           Wrp, Wr, Wq, Wk, Wv, Wo, Wrouter, W1, W2, W3):
    half = DH // 2
    inv_freq = 1.0 / (10000.0 ** (jnp.arange(half, dtype=_f32) * 2.0 / DH))
    ang = positions.astype(_f32)[:, None] * inv_freq[None, :]
    cos = jnp.concatenate([jnp.cos(ang), jnp.cos(ang)], axis=-1)
    sin = jnp.concatenate([jnp.sin(ang), jnp.sin(ang)], axis=-1)

    g1 = ln1_g.reshape(1, D)
    b1 = ln1_b.reshape(1, D)
    g2 = ln2_g.reshape(1, D)
    b2 = ln2_b.reshape(1, D)

    full2 = lambda b, k: (0, 0)
    full3 = lambda b, k: (0, 0, 0)
    blk = lambda b, k: (b, 0, 0)
    x1 = pl.pallas_call(
        _bp_body,
        grid=(B, KH),
        in_specs=[
            pl.BlockSpec((1, T, D), blk),
            pl.BlockSpec((1, D), full2),
            pl.BlockSpec((1, D), full2),
            pl.BlockSpec((D, DH), full2),
            pl.BlockSpec((DH, H), full2),
            pl.BlockSpec((H, D, DH), full3),
            pl.BlockSpec((H, D, DH), full3),
            pl.BlockSpec((H, D, DH), full3),
            pl.BlockSpec((H, DH, D), full3),
            pl.BlockSpec((T, DH), full2),
            pl.BlockSpec((T, DH), full2),
        ],
        out_specs=pl.BlockSpec((1, T, D), blk),
        out_shape=jax.ShapeDtypeStruct((B, T, D), _f32),
        scratch_shapes=[pltpu.VMEM((T, D), _f32),
                        pltpu.SMEM((KH,), jnp.int32),
                        pltpu.SMEM((KH,), _f32)],
        compiler_params=pltpu.CompilerParams(
            dimension_semantics=("arbitrary", "arbitrary")),
    )(x, g1, b1, Wrp.T, Wr.T, Wq, Wk, Wv, Wo, cos, sin)

    x12r = x1.reshape(B * T, D)
    h2, fw = pl.pallas_call(
        _c0_body,
        grid=(NTB,),
        in_specs=[
            pl.BlockSpec((TB, D), lambda t: (t, 0)),
            pl.BlockSpec((1, D), lambda t: (0, 0)),
            pl.BlockSpec((1, D), lambda t: (0, 0)),
            pl.BlockSpec((D, E), lambda t: (0, 0)),
        ],
        out_specs=[
            pl.BlockSpec((TB, D), lambda t: (t, 0)),
            pl.BlockSpec((TB, E), lambda t: (t, 0)),
        ],
        out_shape=[
            jax.ShapeDtypeStruct((B * T, D), _bf),
            jax.ShapeDtypeStruct((B * T, E), _f32),
        ],
    )(x12r, g2, b2, Wrouter.T)

    x12 = x1.reshape(B * T, D)
    out = pl.pallas_call(
        _c1_body,
        grid=(NTB, E),
        in_specs=[
            pl.BlockSpec((TB, D), lambda t, e: (t, 0)),
            pl.BlockSpec((E, D, F), lambda t, e: (0, 0, 0)),
            pl.BlockSpec((E, D, F), lambda t, e: (0, 0, 0)),
            pl.BlockSpec((E, F, D), lambda t, e: (0, 0, 0)),
            pl.BlockSpec((TB, E), lambda t, e: (t, 0)),
            pl.BlockSpec((TB, D), lambda t, e: (t, 0)),
        ],
        out_specs=pl.BlockSpec((TB, D), lambda t, e: (t, 0)),
        out_shape=jax.ShapeDtypeStruct((B * T, D), _f32),
        compiler_params=pltpu.CompilerParams(
            dimension_semantics=("arbitrary", "arbitrary")),
    )(h2, W1.astype(_bf), W3.astype(_bf), W2.astype(_bf), fw, x12)

    return out.reshape(B, T, D)
